# Initial kernel scaffold; baseline (speedup 1.0000x reference)
import functools

import jax
import jax.numpy as jnp
from jax import lax
from jax.experimental import pallas as pl
from jax.experimental.pallas import tpu as pltpu
from jax.experimental.pallas import tpu_sc as plsc

_N = 10000
_D = 128
_G = 64

_NC = 2
_NS = 16
_NW = _NC * _NS
_CHUNK = 128

_N_ACC = 10016
_STRIPE = _N_ACC // _NS

_CNT_ROWS = 640
_N_CNT = _CNT_ROWS * 16
_CNT_IDX_BLKS = _CNT_ROWS // 128
_CNT_STRIPE = _CNT_ROWS // _NS


def _sc_mesh():
    return plsc.VectorSubcoreMesh(core_axis_name="c", subcore_axis_name="s")


def _make_cnt_kernel(e_pad):
    ew = e_pad // _NW
    copies = 8
    per_copy = ew // copies
    assert per_copy * copies == ew and per_copy % 16 == 0 and per_copy % 8 == 0

    @functools.partial(
        pl.kernel,
        out_type=jax.ShapeDtypeStruct((_NC, _CNT_ROWS, 16), jnp.float32),
        mesh=_sc_mesh(),
        scratch_types=[
            pltpu.VMEM((_CNT_ROWS, 16), jnp.float32),
            pltpu.VMEM((per_copy,), jnp.int32),
            pltpu.VMEM((_CNT_IDX_BLKS, 128), jnp.int32),
            pltpu.VMEM_SHARED((_CNT_ROWS, 16), jnp.float32),
        ],
    )
    def cnt_kernel(dst_hbm, out_hbm, cnt_v, dbuf, rowids, cnt_sh):
        c = lax.axis_index("c")
        s = lax.axis_index("s")
        wid = s * _NC + c
        lane = lax.broadcasted_iota(jnp.int32, (16,), 0)
        zero16 = jnp.zeros((16,), jnp.float32)
        ones16 = jnp.ones((16,), jnp.float32)

        def z_body(i, carry):
            cnt_v[i, :] = zero16
            return carry
        lax.fori_loop(0, _CNT_ROWS, z_body, 0)
        for j in range(_CNT_IDX_BLKS):
            for k in range(8):
                rowids[j, pl.ds(k * 16, 16)] = j * 128 + k * 16 + lane

        pltpu.sync_copy(cnt_v.at[pl.ds(s * _CNT_STRIPE, _CNT_STRIPE)],
                        cnt_sh.at[pl.ds(s * _CNT_STRIPE, _CNT_STRIPE)])
        plsc.subcore_barrier()

        def outer(j, carry):
            pltpu.sync_copy(dst_hbm.at[pl.ds(wid * ew + j * per_copy, per_copy)],
                            dbuf)

            def inner(k, c2):
                idx = dbuf[pl.ds(k * 16, 16)]
                row = lax.shift_right_logical(idx, 4)
                ln = lax.bitwise_and(idx, 15)
                plsc.addupdate_scatter(cnt_v, (row, ln), ones16)
                return c2
            lax.fori_loop(0, per_copy // 16, inner, 0)
            return carry
        lax.fori_loop(0, copies, outer, 0)

        for j in range(_CNT_IDX_BLKS):
            pltpu.sync_copy(cnt_v.at[pl.ds(j * 128, 128)],
                            cnt_sh.at[rowids.at[j]], add=True)
        plsc.subcore_barrier()

        pltpu.sync_copy(cnt_sh.at[pl.ds(s * _CNT_STRIPE, _CNT_STRIPE)],
                        out_hbm.at[c].at[pl.ds(s * _CNT_STRIPE, _CNT_STRIPE)])

    return cnt_kernel


def _make_scatter_kernel(e_pad):
    ew = e_pad // _NW
    nchunk = ew // _CHUNK
    assert nchunk * _CHUNK == ew and ew % 8 == 0

    @functools.partial(
        pl.kernel,
        out_type=jax.ShapeDtypeStruct((_NC, _N_ACC, _D), jnp.float32),
        mesh=_sc_mesh(),
        scratch_types=[
            pltpu.VMEM((_CHUNK,), jnp.int32),
            pltpu.VMEM((_CHUNK,), jnp.int32),
            pltpu.VMEM((_CHUNK, _D), jnp.float32),
            pltpu.VMEM_SHARED((_N_ACC, _D), jnp.float32),
            pltpu.SemaphoreType.DMA,
        ],
    )
    def scatter_kernel(hs_hbm, src_hbm, dst_hbm, zeros_hbm, out_hbm,
                       src_v, dst_v, rows_v, acc, sem):
        c = lax.axis_index("c")
        s = lax.axis_index("s")
        wid = s * _NC + c

        pltpu.sync_copy(zeros_hbm.at[pl.ds(s * _STRIPE, _STRIPE)],
                        acc.at[pl.ds(s * _STRIPE, _STRIPE)])
        plsc.subcore_barrier()

        def step(i, carry):
            base = wid * ew + i * _CHUNK
            pltpu.sync_copy(src_hbm.at[pl.ds(base, _CHUNK)], src_v)
            pltpu.sync_copy(dst_hbm.at[pl.ds(base, _CHUNK)], dst_v)
            pltpu.async_copy(hs_hbm.at[src_v], rows_v, sem).wait()
            pltpu.sync_copy(rows_v, acc.at[dst_v], add=True)
            return carry
        lax.fori_loop(0, nchunk, step, 0)
        plsc.subcore_barrier()

        pltpu.sync_copy(acc.at[pl.ds(s * _STRIPE, _STRIPE)],
                        out_hbm.at[c].at[pl.ds(s * _STRIPE, _STRIPE)])

    return scatter_kernel


def _t1_body(cnt_ref, x_ref, w_ref, dinv_ref, s1_ref):
    cnt = cnt_ref[...]
    flat = (cnt[0] + cnt[1]).reshape(_N_CNT)
    deg = flat[:_N] + 1.0
    dinv = lax.rsqrt(deg)[:, None]
    dinv_ref[...] = dinv
    mm = jnp.dot(x_ref[...], w_ref[...], preferred_element_type=jnp.float32)
    s1_ref[...] = dinv * mm


def _tmid_body(p_ref, sprev_ref, dinv_ref, b_ref, w_ref, snext_ref):
    dinv = dinv_ref[...]
    accv = p_ref[0, :_N, :] + p_ref[1, :_N, :] + sprev_ref[...]
    a = jnp.maximum(dinv * accv + b_ref[...], 0.0)
    snext_ref[...] = dinv * jnp.dot(a, w_ref[...],
                                    preferred_element_type=jnp.float32)


def _t4_body(p_ref, sprev_ref, dinv_ref, b_ref, batch_ref, wfc_ref, bfc_ref,
             out_ref):
    dinv = dinv_ref[...]
    accv = p_ref[0, :_N, :] + p_ref[1, :_N, :] + sprev_ref[...]
    a = jnp.maximum(dinv * accv + b_ref[...], 0.0)
    gid = lax.broadcasted_iota(jnp.int32, (_G, _N), 0)
    onehot = (batch_ref[...] == gid).astype(jnp.float32)
    sums = jnp.dot(onehot, a, preferred_element_type=jnp.float32)
    counts = jnp.sum(onehot, axis=1)[:, None]
    pooled = sums / jnp.maximum(counts, 1.0)
    out_ref[...] = jnp.dot(pooled, wfc_ref[...],
                           preferred_element_type=jnp.float32) + bfc_ref[...]


def kernel(x, edge_index, batch, W1, b1, W2, b2, W3, b3, Wfc, bfc):
    e = edge_index.shape[1]
    nchunk_w = -(-e // (_NW * _CHUNK))
    e_pad = _NW * nchunk_w * _CHUNK
    pad = e_pad - e

    src_pad = jnp.concatenate([edge_index[0],
                               jnp.zeros((pad,), jnp.int32)])
    dst_pad = jnp.concatenate([edge_index[1],
                               jnp.full((pad,), _N, jnp.int32)])
    zeros_acc = jnp.zeros((_N_ACC, _D), jnp.float32)

    cnt = _make_cnt_kernel(e_pad)(dst_pad)

    dinv, s1 = pl.pallas_call(
        _t1_body,
        out_shape=(jax.ShapeDtypeStruct((_N, 1), jnp.float32),
                   jax.ShapeDtypeStruct((_N, _D), jnp.float32)),
    )(cnt, x, W1)

    scatter = _make_scatter_kernel(e_pad)

    def mid(s_prev, b_prev, w_next):
        p = scatter(s_prev, src_pad, dst_pad, zeros_acc)
        return pl.pallas_call(
            _tmid_body,
            out_shape=jax.ShapeDtypeStruct((_N, _D), jnp.float32),
        )(p, s_prev, dinv, b_prev.reshape(1, _D), w_next)

    s2 = mid(s1, b1, W2)
    s3 = mid(s2, b2, W3)

    p3 = scatter(s3, src_pad, dst_pad, zeros_acc)
    out = pl.pallas_call(
        _t4_body,
        out_shape=jax.ShapeDtypeStruct((_G, bfc.shape[0]), jnp.float32),
    )(p3, s3, dinv, b3.reshape(1, _D), batch.reshape(1, _N), Wfc,
      bfc.reshape(1, bfc.shape[0]))
    return out


# speedup vs baseline: 9.5910x; 9.5910x over previous
import functools

import jax
import jax.numpy as jnp
from jax import lax
from jax.experimental import pallas as pl
from jax.experimental.pallas import tpu as pltpu
from jax.experimental.pallas import tpu_sc as plsc

_N = 10000
_D = 128
_G = 64

_NC = 2
_NS = 16
_NW = _NC * _NS
_CHUNK = 128

_N_ACC = 10112
_STRIPE = _N_ACC // _NS

_N_CNT = 10240


def _sc_mesh():
    return plsc.VectorSubcoreMesh(core_axis_name="c", subcore_axis_name="s")


def _make_cnt_kernel(e_pad):
    ew = e_pad // _NW
    copies = 8
    per_copy = ew // copies
    assert per_copy * copies == ew and per_copy % 16 == 0 and per_copy % 8 == 0

    @functools.partial(
        pl.kernel,
        out_type=jax.ShapeDtypeStruct((_NW, _N_CNT), jnp.float32),
        mesh=_sc_mesh(),
        scratch_types=[
            pltpu.VMEM((_N_CNT,), jnp.float32),
            pltpu.VMEM((per_copy,), jnp.int32),
        ],
        compiler_params=pltpu.CompilerParams(needs_layout_passes=False),
    )
    def cnt_kernel(dst_hbm, out_hbm, cnt_v, dbuf):
        c = lax.axis_index("c")
        s = lax.axis_index("s")
        wid = s * _NC + c
        zero16 = jnp.zeros((16,), jnp.float32)
        ones16 = jnp.ones((16,), jnp.float32)

        def z_body(i, carry):
            cnt_v[pl.ds(i * 16, 16)] = zero16
            return carry
        lax.fori_loop(0, _N_CNT // 16, z_body, 0)

        def outer(j, carry):
            pltpu.sync_copy(dst_hbm.at[pl.ds(wid * ew + j * per_copy, per_copy)],
                            dbuf)

            def inner(k, c2):
                idx = dbuf[pl.ds(k * 16, 16)]
                plsc.addupdate_scatter(cnt_v, (idx,), ones16)
                return c2
            lax.fori_loop(0, per_copy // 16, inner, 0)
            return carry
        lax.fori_loop(0, copies, outer, 0)

        pltpu.sync_copy(cnt_v, out_hbm.at[wid])

    return cnt_kernel


def _make_scatter_kernel(e_pad):
    ew = e_pad // _NW
    nchunk = ew // _CHUNK
    assert nchunk * _CHUNK == ew and ew % 8 == 0

    @functools.partial(
        pl.kernel,
        out_type=jax.ShapeDtypeStruct((_NC, _N_ACC, _D), jnp.float32),
        mesh=_sc_mesh(),
        scratch_types=[
            pltpu.VMEM((_CHUNK,), jnp.int32),
            pltpu.VMEM((_CHUNK,), jnp.int32),
            pltpu.VMEM((_CHUNK, _D), jnp.float32),
            pltpu.VMEM_SHARED((_N_ACC, _D), jnp.float32),
            pltpu.SemaphoreType.DMA,
        ],
    )
    def scatter_kernel(hs_hbm, src_hbm, dst_hbm, zeros_hbm, out_hbm,
                       src_v, dst_v, rows_v, acc, sem):
        c = lax.axis_index("c")
        s = lax.axis_index("s")
        wid = s * _NC + c

        pltpu.sync_copy(zeros_hbm.at[pl.ds(s * _STRIPE, _STRIPE)],
                        acc.at[pl.ds(s * _STRIPE, _STRIPE)])
        plsc.subcore_barrier()

        def step(i, carry):
            base = wid * ew + i * _CHUNK
            pltpu.sync_copy(src_hbm.at[pl.ds(base, _CHUNK)], src_v)
            pltpu.sync_copy(dst_hbm.at[pl.ds(base, _CHUNK)], dst_v)
            pltpu.async_copy(hs_hbm.at[src_v], rows_v, sem).wait()
            pltpu.sync_copy(rows_v, acc.at[dst_v], add=True)
            return carry
        lax.fori_loop(0, nchunk, step, 0)
        plsc.subcore_barrier()

        pltpu.sync_copy(acc.at[pl.ds(s * _STRIPE, _STRIPE)],
                        out_hbm.at[c].at[pl.ds(s * _STRIPE, _STRIPE)])

    return scatter_kernel


def _t1_body(cnt_ref, x_ref, w_ref, dinv_ref, s1_ref):
    flat = jnp.sum(cnt_ref[...], axis=0)
    deg = flat[:_N] + 1.0
    dinv = lax.rsqrt(deg)[:, None]
    dinv_ref[...] = dinv
    mm = jnp.dot(x_ref[...], w_ref[...], preferred_element_type=jnp.float32)
    s1_ref[...] = dinv * mm


def _tmid_body(p_ref, sprev_ref, dinv_ref, b_ref, w_ref, snext_ref):
    dinv = dinv_ref[...]
    accv = p_ref[0, :_N, :] + p_ref[1, :_N, :] + sprev_ref[...]
    a = jnp.maximum(dinv * accv + b_ref[...], 0.0)
    snext_ref[...] = dinv * jnp.dot(a, w_ref[...],
                                    preferred_element_type=jnp.float32)


def _t4_body(p_ref, sprev_ref, dinv_ref, b_ref, batch_ref, wfc_ref, bfc_ref,
             out_ref):
    dinv = dinv_ref[...]
    accv = p_ref[0, :_N, :] + p_ref[1, :_N, :] + sprev_ref[...]
    a = jnp.maximum(dinv * accv + b_ref[...], 0.0)
    gid = lax.broadcasted_iota(jnp.int32, (_G, _N), 0)
    onehot = (batch_ref[...] == gid).astype(jnp.float32)
    sums = jnp.dot(onehot, a, preferred_element_type=jnp.float32)
    counts = jnp.sum(onehot, axis=1)[:, None]
    pooled = sums / jnp.maximum(counts, 1.0)
    out_ref[...] = jnp.dot(pooled, wfc_ref[...],
                           preferred_element_type=jnp.float32) + bfc_ref[...]


def kernel(x, edge_index, batch, W1, b1, W2, b2, W3, b3, Wfc, bfc):
    e = edge_index.shape[1]
    nchunk_w = -(-e // (_NW * _CHUNK))
    e_pad = _NW * nchunk_w * _CHUNK
    pad = e_pad - e

    src_pad = jnp.concatenate([edge_index[0],
                               jnp.zeros((pad,), jnp.int32)])
    dst_pad = jnp.concatenate([edge_index[1],
                               jnp.full((pad,), _N, jnp.int32)])
    zeros_acc = jnp.zeros((_N_ACC, _D), jnp.float32)

    cnt = _make_cnt_kernel(e_pad)(dst_pad)

    dinv, s1 = pl.pallas_call(
        _t1_body,
        out_shape=(jax.ShapeDtypeStruct((_N, 1), jnp.float32),
                   jax.ShapeDtypeStruct((_N, _D), jnp.float32)),
    )(cnt, x, W1)

    scatter = _make_scatter_kernel(e_pad)

    def mid(s_prev, b_prev, w_next):
        p = scatter(s_prev, src_pad, dst_pad, zeros_acc)
        return pl.pallas_call(
            _tmid_body,
            out_shape=jax.ShapeDtypeStruct((_N, _D), jnp.float32),
        )(p, s_prev, dinv, b_prev.reshape(1, _D), w_next)

    s2 = mid(s1, b1, W2)
    s3 = mid(s2, b2, W3)

    p3 = scatter(s3, src_pad, dst_pad, zeros_acc)
    out = pl.pallas_call(
        _t4_body,
        out_shape=jax.ShapeDtypeStruct((_G, bfc.shape[0]), jnp.float32),
    )(p3, s3, dinv, b3.reshape(1, _D), batch.reshape(1, _N), Wfc,
      bfc.reshape(1, bfc.shape[0]))
    return out
